# baseline (speedup 1.0000x reference)
import jax
import jax.numpy as jnp
from jax import lax
from jax.experimental import pallas as pl

NUM_SERIES_C = 128
B_BLK = 64


def _norm_body(i_ref, loc_ref, scale_ref, x_ref, o_ref):
    iv = i_ref[0, 0, :]
    onehot = iv[:, None] == lax.broadcasted_iota(
        jnp.int32, (B_BLK, NUM_SERIES_C), 1
    )
    loc_row = loc_ref[0, :]
    scale_row = scale_ref[0, :]
    lg = jnp.sum(jnp.where(onehot, loc_row[None, :], 0.0), axis=1)
    sg = jnp.sum(jnp.where(onehot, scale_row[None, :], 0.0), axis=1)
    rg = 1.0 / sg
    o_ref[...] = (x_ref[...] - lg[:, None]) * rg[:, None]


def kernel(x, i, loc, scale):
    bs, num_patch, out_len = x.shape
    row = num_patch * out_len
    nblk = bs // B_BLK
    x2 = x.reshape(bs, row)
    i3 = i.reshape(nblk, 1, B_BLK)
    loc2 = loc.reshape(1, -1)
    scale2 = scale.reshape(1, -1)

    out = pl.pallas_call(
        _norm_body,
        grid=(nblk,),
        in_specs=[
            pl.BlockSpec((1, 1, B_BLK), lambda b: (b, 0, 0)),
            pl.BlockSpec((1, NUM_SERIES_C), lambda b: (0, 0)),
            pl.BlockSpec((1, NUM_SERIES_C), lambda b: (0, 0)),
            pl.BlockSpec((B_BLK, row), lambda b: (b, 0)),
        ],
        out_specs=pl.BlockSpec((B_BLK, row), lambda b: (b, 0)),
        out_shape=jax.ShapeDtypeStruct((bs, row), x.dtype),
    )(i3, loc2, scale2, x2)
    return out.reshape(bs, num_patch, out_len)


# speedup vs baseline: 1.0449x; 1.0449x over previous
import jax
import jax.numpy as jnp
from jax import lax
from jax.experimental import pallas as pl
from jax.experimental.pallas import tpu as pltpu

NUM_SERIES_C = 128
C = 32
NBUF = 8


def _body(i_ref, loc_ref, scale_ref, x_hbm, o_hbm,
          in_buf, out_buf, sem_in, sem_out):
    num_chunks = x_hbm.shape[0] // C

    def in_copy(c, j):
        return pltpu.make_async_copy(
            x_hbm.at[pl.ds(c * C, C), :],
            in_buf.at[pl.ds(j * C, C), :],
            sem_in.at[j],
        )

    def out_copy(c, j):
        return pltpu.make_async_copy(
            out_buf.at[pl.ds(j * C, C), :],
            o_hbm.at[pl.ds(c * C, C), :],
            sem_out.at[j],
        )

    for j in range(NBUF):
        in_copy(j, j).start()

    loc_row = loc_ref[0, :]
    scale_row = scale_ref[0, :]

    def step(c, _):
        j = lax.rem(c, NBUF)
        in_copy(c, j).wait()

        iv = i_ref[c, :]
        onehot = iv[:, None] == lax.broadcasted_iota(
            jnp.int32, (C, NUM_SERIES_C), 1
        )
        lg = jnp.sum(jnp.where(onehot, loc_row[None, :], 0.0), axis=1)
        sg = jnp.sum(jnp.where(onehot, scale_row[None, :], 0.0), axis=1)
        rg = 1.0 / sg

        @pl.when(c >= NBUF)
        def _():
            out_copy(c - NBUF, j).wait()

        xin = in_buf[pl.ds(j * C, C), :]
        out_buf[pl.ds(j * C, C), :] = (xin - lg[:, None]) * rg[:, None]
        out_copy(c, j).start()

        @pl.when(c + NBUF < num_chunks)
        def _():
            in_copy(c + NBUF, j).start()

        return _

    lax.fori_loop(0, num_chunks, step, None)

    for j in range(NBUF):
        out_copy(num_chunks - NBUF + j, j).wait()


def kernel(x, i, loc, scale):
    bs, num_patch, out_len = x.shape
    row = num_patch * out_len
    num_chunks = bs // C
    x2 = x.reshape(bs, row)
    i2 = i.reshape(num_chunks, C)
    loc2 = loc.reshape(1, -1)
    scale2 = scale.reshape(1, -1)

    out = pl.pallas_call(
        _body,
        in_specs=[
            pl.BlockSpec(memory_space=pltpu.MemorySpace.VMEM),
            pl.BlockSpec(memory_space=pltpu.MemorySpace.VMEM),
            pl.BlockSpec(memory_space=pltpu.MemorySpace.VMEM),
            pl.BlockSpec(memory_space=pltpu.MemorySpace.HBM),
        ],
        out_specs=pl.BlockSpec(memory_space=pltpu.MemorySpace.HBM),
        out_shape=jax.ShapeDtypeStruct((bs, row), x.dtype),
        scratch_shapes=[
            pltpu.VMEM((NBUF * C, row), x.dtype),
            pltpu.VMEM((NBUF * C, row), x.dtype),
            pltpu.SemaphoreType.DMA((NBUF,)),
            pltpu.SemaphoreType.DMA((NBUF,)),
        ],
    )(i2, loc2, scale2, x2)
    return out.reshape(bs, num_patch, out_len)


# speedup vs baseline: 3.5321x; 3.3804x over previous
import jax
import jax.numpy as jnp
from jax import lax
from jax.experimental import pallas as pl
from jax.experimental.pallas import tpu as pltpu

NUM_SERIES_C = 128
C = 32
NBUF = 8


def _body(i_ref, loc_ref, scale_ref, x_hbm, o_hbm,
          in_buf, out_buf, sem_in, sem_out):
    num_chunks = x_hbm.shape[0] // C

    def in_copy(c, j):
        return pltpu.make_async_copy(
            x_hbm.at[pl.ds(c * C, C), :, :],
            in_buf.at[pl.ds(j * C, C), :, :],
            sem_in.at[j],
        )

    def out_copy(c, j):
        return pltpu.make_async_copy(
            out_buf.at[pl.ds(j * C, C), :, :],
            o_hbm.at[pl.ds(c * C, C), :, :],
            sem_out.at[j],
        )

    for j in range(NBUF):
        in_copy(j, j).start()

    loc_row = loc_ref[0, :]
    scale_row = scale_ref[0, :]

    def step(c, _):
        j = lax.rem(c, NBUF)
        in_copy(c, j).wait()

        iv = i_ref[c, :]
        onehot = iv[:, None] == lax.broadcasted_iota(
            jnp.int32, (C, NUM_SERIES_C), 1
        )
        lg = jnp.sum(jnp.where(onehot, loc_row[None, :], 0.0), axis=1)
        sg = jnp.sum(jnp.where(onehot, scale_row[None, :], 0.0), axis=1)
        rg = 1.0 / sg

        @pl.when(c >= NBUF)
        def _():
            out_copy(c - NBUF, j).wait()

        xin = in_buf[pl.ds(j * C, C), :, :]
        out_buf[pl.ds(j * C, C), :, :] = (
            xin - lg[:, None, None]
        ) * rg[:, None, None]
        out_copy(c, j).start()

        @pl.when(c + NBUF < num_chunks)
        def _():
            in_copy(c + NBUF, j).start()

        return _

    lax.fori_loop(0, num_chunks, step, None)

    for j in range(NBUF):
        out_copy(num_chunks - NBUF + j, j).wait()


def kernel(x, i, loc, scale):
    bs, num_patch, out_len = x.shape
    num_chunks = bs // C
    i2 = i.reshape(num_chunks, C)
    loc2 = loc.reshape(1, -1)
    scale2 = scale.reshape(1, -1)

    return pl.pallas_call(
        _body,
        in_specs=[
            pl.BlockSpec(memory_space=pltpu.MemorySpace.VMEM),
            pl.BlockSpec(memory_space=pltpu.MemorySpace.VMEM),
            pl.BlockSpec(memory_space=pltpu.MemorySpace.VMEM),
            pl.BlockSpec(memory_space=pltpu.MemorySpace.HBM),
        ],
        out_specs=pl.BlockSpec(memory_space=pltpu.MemorySpace.HBM),
        out_shape=jax.ShapeDtypeStruct((bs, num_patch, out_len), x.dtype),
        scratch_shapes=[
            pltpu.VMEM((NBUF * C, num_patch, out_len), x.dtype),
            pltpu.VMEM((NBUF * C, num_patch, out_len), x.dtype),
            pltpu.SemaphoreType.DMA((NBUF,)),
            pltpu.SemaphoreType.DMA((NBUF,)),
        ],
    )(i2, loc2, scale2, x)
